# baseline (speedup 1.0000x reference)
import functools

import jax
import jax.numpy as jnp
from jax.experimental import pallas as pl
from jax.experimental.pallas import tpu as pltpu

R_HASH = 0.1
M_POW = 5
TABLES = 10.0
U_SCALE = 0.83


def _norms_kernel(w_ref, norms_ref, mx_ref):
    i = pl.program_id(0)
    w = w_ref[...]
    n = jnp.sqrt(jnp.sum(w * w, axis=1, keepdims=True))
    norms_ref[...] = n

    @pl.when(i == 0)
    def _init():
        mx_ref[...] = jnp.full_like(mx_ref, -jnp.inf)

    mx_ref[...] = jnp.maximum(mx_ref[...], jnp.max(n))


def _row_norms(Wp, blk):
    rows = Wp.shape[0]
    grid = rows // blk
    norms, mx = pl.pallas_call(
        _norms_kernel,
        grid=(grid,),
        in_specs=[pl.BlockSpec((blk, Wp.shape[1]), lambda i: (i, 0))],
        out_specs=[
            pl.BlockSpec((blk, 1), lambda i: (i, 0)),
            pl.BlockSpec((1, 1), lambda i: (0, 0)),
        ],
        out_shape=[
            jax.ShapeDtypeStruct((rows, 1), jnp.float32),
            jax.ShapeDtypeStruct((1, 1), jnp.float32),
        ],
    )(Wp)
    return norms, mx


def _whash_kernel(w_ref, norms_ref, mx_ref, ad_ref, at_ref, hb_ref,
                  hw_ref, *, nvalid, blk):
    i = pl.program_id(0)
    scale = U_SCALE / (mx_ref[0, 0] + 1e-12)
    wp = (w_ref[...] * scale).astype(jnp.bfloat16)
    wdot = jax.lax.dot_general(
        wp, ad_ref[...].astype(jnp.bfloat16), (((1,), (0,)), ((), ())),
        preferred_element_type=jnp.float32)
    sn = norms_ref[...] * scale
    powsum = jnp.zeros_like(sn)
    p = sn
    for k in range(M_POW):
        p = p * p
        term = p.astype(jnp.bfloat16).astype(jnp.float32) * (
            at_ref[k, 0].astype(jnp.bfloat16).astype(jnp.float32))
        powsum = powsum + term
    v = (wdot + powsum + hb_ref[0, 0]) / R_HASH
    hw = jnp.mod(jnp.floor(v), TABLES)
    ridx = jax.lax.broadcasted_iota(jnp.int32, hw.shape, 0) + i * blk
    hw = jnp.where(ridx < nvalid, hw, -1.0)
    hw_ref[...] = hw


def _weight_hash(Wp, norms, mx, ad, at, hb, nvalid, blk):
    rows, d = Wp.shape
    grid = rows // blk
    hw = pl.pallas_call(
        functools.partial(_whash_kernel, nvalid=nvalid, blk=blk),
        grid=(grid,),
        in_specs=[
            pl.BlockSpec((blk, d), lambda i: (i, 0)),
            pl.BlockSpec((blk, 1), lambda i: (i, 0)),
            pl.BlockSpec((1, 1), lambda i: (0, 0)),
            pl.BlockSpec((d, 1), lambda i: (0, 0)),
            pl.BlockSpec((8, 1), lambda i: (0, 0)),
            pl.BlockSpec((1, 1), lambda i: (0, 0)),
        ],
        out_specs=pl.BlockSpec((blk, 1), lambda i: (i, 0)),
        out_shape=jax.ShapeDtypeStruct((rows, 1), jnp.float32),
    )(Wp, norms, mx, ad, at, hb)
    return hw


def _qhash_kernel(x_ref, ad_ref, at_ref, hb_ref, hq_ref):
    x = x_ref[...]
    nrm = jnp.sqrt(jnp.sum(x * x, axis=1, keepdims=True))
    xn = (x / (nrm + 1e-8)).astype(jnp.bfloat16)
    qdot = jax.lax.dot_general(
        xn, ad_ref[...].astype(jnp.bfloat16), (((1,), (0,)), ((), ())),
        preferred_element_type=jnp.float32)
    tailsum = 0.5 * jnp.sum(at_ref[...].astype(jnp.bfloat16).astype(jnp.float32))
    v = (qdot + tailsum + hb_ref[0, 0]) / R_HASH
    hq_ref[...] = jnp.mod(jnp.floor(v), TABLES)


def _query_hash(x, ad, at, hb, blk):
    rows, d = x.shape
    grid = rows // blk
    hq = pl.pallas_call(
        _qhash_kernel,
        grid=(grid,),
        in_specs=[
            pl.BlockSpec((blk, d), lambda i: (i, 0)),
            pl.BlockSpec((d, 1), lambda i: (0, 0)),
            pl.BlockSpec((8, 1), lambda i: (0, 0)),
            pl.BlockSpec((1, 1), lambda i: (0, 0)),
        ],
        out_specs=pl.BlockSpec((blk, 1), lambda i: (i, 0)),
        out_shape=jax.ShapeDtypeStruct((rows, 1), jnp.float32),
    )(x, ad, at, hb)
    return hq


def _masked_linear_kernel(x_ref, w_ref, b_ref, hq_ref, hw_ref, out_ref):
    x = x_ref[...]
    w = w_ref[...]
    xh = x.astype(jnp.bfloat16)
    wh = w.astype(jnp.bfloat16)
    xl = (x - xh.astype(jnp.float32)).astype(jnp.bfloat16)
    wl = (w - wh.astype(jnp.float32)).astype(jnp.bfloat16)
    dims = (((1,), (1,)), ((), ()))
    dense = jax.lax.dot_general(xh, wh, dims,
                                preferred_element_type=jnp.float32)
    dense = dense + jax.lax.dot_general(xh, wl, dims,
                                        preferred_element_type=jnp.float32)
    dense = dense + jax.lax.dot_general(xl, wh, dims,
                                        preferred_element_type=jnp.float32)
    dense = dense + b_ref[...]
    mask = hq_ref[...] == hw_ref[...]
    out_ref[...] = jnp.where(mask, jnp.maximum(dense, 0.0), 0.0)


def _masked_linear(x, Wp, b, hq, hw, bm, bn):
    B, d = x.shape
    H = Wp.shape[0]
    out = pl.pallas_call(
        _masked_linear_kernel,
        grid=(B // bm, H // bn),
        in_specs=[
            pl.BlockSpec((bm, d), lambda i, j: (i, 0)),
            pl.BlockSpec((bn, d), lambda i, j: (j, 0)),
            pl.BlockSpec((1, bn), lambda i, j: (0, j)),
            pl.BlockSpec((bm, 1), lambda i, j: (i, 0)),
            pl.BlockSpec((1, bn), lambda i, j: (0, j)),
        ],
        out_specs=pl.BlockSpec((bm, bn), lambda i, j: (i, j)),
        out_shape=jax.ShapeDtypeStruct((B, H), jnp.float32),
    )(x, Wp, b, hq, hw)
    return out


def _masked_linear_out_kernel(h_ref, w_ref, b_ref, hq_ref, hw_ref,
                              wo_ref, bo_ref, out_ref):
    dense = jax.lax.dot_general(
        h_ref[...].astype(jnp.bfloat16), w_ref[...].astype(jnp.bfloat16),
        (((1,), (1,)), ((), ())),
        preferred_element_type=jnp.float32)
    dense = dense + b_ref[...]
    mask = hq_ref[...] == hw_ref[...]
    h2 = jnp.where(mask, jnp.maximum(dense, 0.0), 0.0)
    out = jax.lax.dot_general(
        h2.astype(jnp.bfloat16), wo_ref[...].astype(jnp.bfloat16),
        (((1,), (1,)), ((), ())),
        preferred_element_type=jnp.float32)
    out_ref[...] = out + bo_ref[...]


def _masked_linear_out(h, Wp, b, hq, hw, Wo, bo, bm):
    B, Hp = h.shape
    Op = Wo.shape[0]
    out = pl.pallas_call(
        _masked_linear_out_kernel,
        grid=(B // bm,),
        in_specs=[
            pl.BlockSpec((bm, Hp), lambda i: (i, 0)),
            pl.BlockSpec((Hp, Hp), lambda i: (0, 0)),
            pl.BlockSpec((1, Hp), lambda i: (0, 0)),
            pl.BlockSpec((bm, 1), lambda i: (i, 0)),
            pl.BlockSpec((1, Hp), lambda i: (0, 0)),
            pl.BlockSpec((Op, Hp), lambda i: (0, 0)),
            pl.BlockSpec((1, Op), lambda i: (0, 0)),
        ],
        out_specs=pl.BlockSpec((bm, Op), lambda i: (i, 0)),
        out_shape=jax.ShapeDtypeStruct((B, Op), jnp.float32),
    )(h, Wp, b, hq, hw, Wo, bo)
    return out


def kernel(x, W1, b1, W2, b2, Wout, bout, a1, hb1, a2, hb2):
    B, D = x.shape
    H = W1.shape[0]
    O = Wout.shape[0]
    Hp = 1024
    Op = 16

    W1p = jnp.pad(W1, ((0, Hp - H), (0, 0)))
    b1p = jnp.pad(b1, (0, Hp - H)).reshape(1, Hp)
    W2p = jnp.pad(W2, ((0, Hp - H), (0, Hp - H)))
    b2p = jnp.pad(b2, (0, Hp - H)).reshape(1, Hp)
    Woutp = jnp.pad(Wout, ((0, Op - O), (0, Hp - H)))
    boutp = jnp.pad(bout, (0, Op - O)).reshape(1, Op)

    a1d = a1[:D].reshape(D, 1)
    a1t = jnp.pad(a1[D:], (0, 8 - M_POW)).reshape(8, 1)
    a2d = jnp.pad(a2[:H], (0, Hp - H)).reshape(Hp, 1)
    a2t = jnp.pad(a2[H:], (0, 8 - M_POW)).reshape(8, 1)
    hb1r = hb1.reshape(1, 1)
    hb2r = hb2.reshape(1, 1)

    n1, mx1 = _row_norms(W1p, 128)
    hw1 = _weight_hash(W1p, n1, mx1, a1d, a1t, hb1r, H, 128)
    hq1 = _query_hash(x, a1d, a1t, hb1r, 256)
    h1 = _masked_linear(x, W1p, b1p, hq1, hw1.reshape(1, Hp), 512, 1024)

    n2, mx2 = _row_norms(W2p, 128)
    hw2 = _weight_hash(W2p, n2, mx2, a2d, a2t, hb2r, H, 128)
    hq2 = _query_hash(h1, a2d, a2t, hb2r, 256)
    out = _masked_linear_out(h1, W2p, b2p, hq2, hw2.reshape(1, Hp),
                             Woutp, boutp, 256)
    return out[:, :O]


# speedup vs baseline: 1.3171x; 1.3171x over previous
import functools

import jax
import jax.numpy as jnp
from jax.experimental import pallas as pl

R_HASH = 0.1
M_POW = 5
TABLES = 10.0
U_SCALE = 0.83


def _norms_kernel(w_ref, norms_ref, mx_ref):
    i = pl.program_id(0)
    w = w_ref[...]
    n = jnp.sqrt(jnp.sum(w * w, axis=1, keepdims=True))
    norms_ref[...] = n

    @pl.when(i == 0)
    def _init():
        mx_ref[...] = jnp.full_like(mx_ref, -jnp.inf)

    mx_ref[...] = jnp.maximum(mx_ref[...], jnp.max(n))


def _row_norms(Wp, blk):
    rows = Wp.shape[0]
    grid = rows // blk
    norms, mx = pl.pallas_call(
        _norms_kernel,
        grid=(grid,),
        in_specs=[pl.BlockSpec((blk, Wp.shape[1]), lambda i: (i, 0))],
        out_specs=[
            pl.BlockSpec((blk, 1), lambda i: (i, 0)),
            pl.BlockSpec((1, 1), lambda i: (0, 0)),
        ],
        out_shape=[
            jax.ShapeDtypeStruct((rows, 1), jnp.float32),
            jax.ShapeDtypeStruct((1, 1), jnp.float32),
        ],
    )(Wp)
    return norms, mx


def _whash_kernel(w_ref, norms_ref, mx_ref, ad_ref, at_ref, hb_ref,
                  hw_ref, *, nvalid, blk):
    i = pl.program_id(0)
    scale = U_SCALE / (mx_ref[0, 0] + 1e-12)
    wp = (w_ref[...] * scale).astype(jnp.bfloat16)
    wdot = jax.lax.dot_general(
        wp, ad_ref[...].astype(jnp.bfloat16), (((1,), (0,)), ((), ())),
        preferred_element_type=jnp.float32)
    sn = norms_ref[...] * scale
    powsum = jnp.zeros_like(sn)
    p = sn
    for k in range(M_POW):
        p = p * p
        term = p.astype(jnp.bfloat16).astype(jnp.float32) * (
            at_ref[k, 0].astype(jnp.bfloat16).astype(jnp.float32))
        powsum = powsum + term
    v = (wdot + powsum + hb_ref[0, 0]) / R_HASH
    hw = jnp.mod(jnp.floor(v), TABLES)
    ridx = jax.lax.broadcasted_iota(jnp.int32, hw.shape, 0) + i * blk
    hw = jnp.where(ridx < nvalid, hw, -1.0)
    hw_ref[...] = hw


def _weight_hash(Wp, norms, mx, ad, at, hb, nvalid, blk):
    rows, d = Wp.shape
    grid = rows // blk
    hw = pl.pallas_call(
        functools.partial(_whash_kernel, nvalid=nvalid, blk=blk),
        grid=(grid,),
        in_specs=[
            pl.BlockSpec((blk, d), lambda i: (i, 0)),
            pl.BlockSpec((blk, 1), lambda i: (i, 0)),
            pl.BlockSpec((1, 1), lambda i: (0, 0)),
            pl.BlockSpec((d, 1), lambda i: (0, 0)),
            pl.BlockSpec((8, 1), lambda i: (0, 0)),
            pl.BlockSpec((1, 1), lambda i: (0, 0)),
        ],
        out_specs=pl.BlockSpec((blk, 1), lambda i: (i, 0)),
        out_shape=jax.ShapeDtypeStruct((rows, 1), jnp.float32),
    )(Wp, norms, mx, ad, at, hb)
    return hw


def _qhash_kernel(x_ref, ad_ref, at_ref, hb_ref, hq_ref):
    x = x_ref[...]
    nrm = jnp.sqrt(jnp.sum(x * x, axis=1, keepdims=True))
    xn = (x / (nrm + 1e-8)).astype(jnp.bfloat16)
    qdot = jax.lax.dot_general(
        xn, ad_ref[...].astype(jnp.bfloat16), (((1,), (0,)), ((), ())),
        preferred_element_type=jnp.float32)
    tailsum = 0.5 * jnp.sum(at_ref[...].astype(jnp.bfloat16).astype(jnp.float32))
    v = (qdot + tailsum + hb_ref[0, 0]) / R_HASH
    hq_ref[...] = jnp.mod(jnp.floor(v), TABLES)


def _query_hash(x, ad, at, hb, blk):
    rows, d = x.shape
    grid = rows // blk
    hq = pl.pallas_call(
        _qhash_kernel,
        grid=(grid,),
        in_specs=[
            pl.BlockSpec((blk, d), lambda i: (i, 0)),
            pl.BlockSpec((d, 1), lambda i: (0, 0)),
            pl.BlockSpec((8, 1), lambda i: (0, 0)),
            pl.BlockSpec((1, 1), lambda i: (0, 0)),
        ],
        out_specs=pl.BlockSpec((blk, 1), lambda i: (i, 0)),
        out_shape=jax.ShapeDtypeStruct((rows, 1), jnp.float32),
    )(x, ad, at, hb)
    return hq


def _masked_linear_kernel(x_ref, w_ref, b_ref, hq_ref, hw_ref, out_ref):
    dense = jax.lax.dot_general(
        x_ref[...], w_ref[...], (((1,), (1,)), ((), ())),
        preferred_element_type=jnp.float32)
    dense = dense + b_ref[...]
    mask = hq_ref[...] == hw_ref[...]
    out_ref[...] = jnp.where(mask, jnp.maximum(dense, 0.0), 0.0)


def _masked_linear(x, Wp, b, hq, hw, bm, bn):
    B, d = x.shape
    H = Wp.shape[0]
    out = pl.pallas_call(
        _masked_linear_kernel,
        grid=(B // bm, H // bn),
        in_specs=[
            pl.BlockSpec((bm, d), lambda i, j: (i, 0)),
            pl.BlockSpec((bn, d), lambda i, j: (j, 0)),
            pl.BlockSpec((1, bn), lambda i, j: (0, j)),
            pl.BlockSpec((bm, 1), lambda i, j: (i, 0)),
            pl.BlockSpec((1, bn), lambda i, j: (0, j)),
        ],
        out_specs=pl.BlockSpec((bm, bn), lambda i, j: (i, j)),
        out_shape=jax.ShapeDtypeStruct((B, H), jnp.float32),
    )(x, Wp, b, hq, hw)
    return out


def _masked_linear_out_kernel(h_ref, w_ref, b_ref, hq_ref, hw_ref,
                              wo_ref, bo_ref, out_ref):
    dense = jax.lax.dot_general(
        h_ref[...].astype(jnp.bfloat16), w_ref[...].astype(jnp.bfloat16),
        (((1,), (1,)), ((), ())),
        preferred_element_type=jnp.float32)
    dense = dense + b_ref[...]
    mask = hq_ref[...] == hw_ref[...]
    h2 = jnp.where(mask, jnp.maximum(dense, 0.0), 0.0)
    out = jax.lax.dot_general(
        h2.astype(jnp.bfloat16), wo_ref[...].astype(jnp.bfloat16),
        (((1,), (1,)), ((), ())),
        preferred_element_type=jnp.float32)
    out_ref[...] = out + bo_ref[...]


def _masked_linear_out(h, Wp, b, hq, hw, Wo, bo, bm):
    B, Hp = h.shape
    Op = Wo.shape[0]
    out = pl.pallas_call(
        _masked_linear_out_kernel,
        grid=(B // bm,),
        in_specs=[
            pl.BlockSpec((bm, Hp), lambda i: (i, 0)),
            pl.BlockSpec((Hp, Hp), lambda i: (0, 0)),
            pl.BlockSpec((1, Hp), lambda i: (0, 0)),
            pl.BlockSpec((bm, 1), lambda i: (i, 0)),
            pl.BlockSpec((1, Hp), lambda i: (0, 0)),
            pl.BlockSpec((Op, Hp), lambda i: (0, 0)),
            pl.BlockSpec((1, Op), lambda i: (0, 0)),
        ],
        out_specs=pl.BlockSpec((bm, Op), lambda i: (i, 0)),
        out_shape=jax.ShapeDtypeStruct((B, Op), jnp.float32),
    )(h, Wp, b, hq, hw, Wo, bo)
    return out


def kernel(x, W1, b1, W2, b2, Wout, bout, a1, hb1, a2, hb2):
    B, D = x.shape
    H = W1.shape[0]
    O = Wout.shape[0]
    Hp = 1024
    Op = 16

    W1p = jnp.pad(W1, ((0, Hp - H), (0, 0)))
    b1p = jnp.pad(b1, (0, Hp - H)).reshape(1, Hp)
    W2p = jnp.pad(W2, ((0, Hp - H), (0, Hp - H)))
    b2p = jnp.pad(b2, (0, Hp - H)).reshape(1, Hp)
    Woutp = jnp.pad(Wout, ((0, Op - O), (0, Hp - H)))
    boutp = jnp.pad(bout, (0, Op - O)).reshape(1, Op)

    a1d = a1[:D].reshape(D, 1)
    a1t = jnp.pad(a1[D:], (0, 8 - M_POW)).reshape(8, 1)
    a2d = jnp.pad(a2[:H], (0, Hp - H)).reshape(Hp, 1)
    a2t = jnp.pad(a2[H:], (0, 8 - M_POW)).reshape(8, 1)
    hb1r = hb1.reshape(1, 1)
    hb2r = hb2.reshape(1, 1)

    n1, mx1 = _row_norms(W1p, 128)
    hw1 = _weight_hash(W1p, n1, mx1, a1d, a1t, hb1r, H, 128)
    hq1 = _query_hash(x, a1d, a1t, hb1r, 256)
    h1 = _masked_linear(x, W1p, b1p, hq1, hw1.reshape(1, Hp), 512, 1024)

    n2, mx2 = _row_norms(W2p, 128)
    hw2 = _weight_hash(W2p, n2, mx2, a2d, a2t, hb2r, H, 128)
    hq2 = _query_hash(h1, a2d, a2t, hb2r, 256)
    out = _masked_linear_out(h1, W2p, b2p, hq2, hw2.reshape(1, Hp),
                             Woutp, boutp, 256)
    return out[:, :O]


# speedup vs baseline: 1.6145x; 1.2258x over previous
import functools

import jax
import jax.numpy as jnp
from jax.experimental import pallas as pl

R_HASH = 0.1
M_POW = 5
TABLES = 10.0
U_SCALE = 0.83


def _norms_kernel(w_ref, norms_ref, mx_ref):
    i = pl.program_id(0)
    w = w_ref[...]
    n = jnp.sqrt(jnp.sum(w * w, axis=1, keepdims=True))
    norms_ref[...] = n

    @pl.when(i == 0)
    def _init():
        mx_ref[...] = jnp.full_like(mx_ref, -jnp.inf)

    mx_ref[...] = jnp.maximum(mx_ref[...], jnp.max(n))


def _row_norms(Wp, blk):
    rows = Wp.shape[0]
    grid = rows // blk
    norms, mx = pl.pallas_call(
        _norms_kernel,
        grid=(grid,),
        in_specs=[pl.BlockSpec((blk, Wp.shape[1]), lambda i: (i, 0))],
        out_specs=[
            pl.BlockSpec((blk, 1), lambda i: (i, 0)),
            pl.BlockSpec((1, 1), lambda i: (0, 0)),
        ],
        out_shape=[
            jax.ShapeDtypeStruct((rows, 1), jnp.float32),
            jax.ShapeDtypeStruct((1, 1), jnp.float32),
        ],
    )(Wp)
    return norms, mx


def _whash_kernel(w_ref, norms_ref, mx_ref, ad_ref, at_ref, hb_ref,
                  hw_ref, *, nvalid, blk):
    i = pl.program_id(0)
    scale = U_SCALE / (mx_ref[0, 0] + 1e-12)
    wp = (w_ref[...] * scale).astype(jnp.bfloat16)
    wdot = jax.lax.dot_general(
        wp, ad_ref[...].astype(jnp.bfloat16), (((1,), (0,)), ((), ())),
        preferred_element_type=jnp.float32)
    sn = norms_ref[...] * scale
    powsum = jnp.zeros_like(sn)
    p = sn
    for k in range(M_POW):
        p = p * p
        term = p.astype(jnp.bfloat16).astype(jnp.float32) * (
            at_ref[k, 0].astype(jnp.bfloat16).astype(jnp.float32))
        powsum = powsum + term
    v = (wdot + powsum + hb_ref[0, 0]) / R_HASH
    hw = jnp.mod(jnp.floor(v), TABLES)
    ridx = jax.lax.broadcasted_iota(jnp.int32, hw.shape, 0) + i * blk
    hw = jnp.where(ridx < nvalid, hw, -1.0)
    hw_ref[...] = hw


def _weight_hash(Wp, norms, mx, ad, at, hb, nvalid, blk):
    rows, d = Wp.shape
    grid = rows // blk
    hw = pl.pallas_call(
        functools.partial(_whash_kernel, nvalid=nvalid, blk=blk),
        grid=(grid,),
        in_specs=[
            pl.BlockSpec((blk, d), lambda i: (i, 0)),
            pl.BlockSpec((blk, 1), lambda i: (i, 0)),
            pl.BlockSpec((1, 1), lambda i: (0, 0)),
            pl.BlockSpec((d, 1), lambda i: (0, 0)),
            pl.BlockSpec((8, 1), lambda i: (0, 0)),
            pl.BlockSpec((1, 1), lambda i: (0, 0)),
        ],
        out_specs=pl.BlockSpec((blk, 1), lambda i: (i, 0)),
        out_shape=jax.ShapeDtypeStruct((rows, 1), jnp.float32),
    )(Wp, norms, mx, ad, at, hb)
    return hw


def _qhash_kernel(x_ref, ad_ref, at_ref, hb_ref, hq_ref):
    x = x_ref[...]
    nrm = jnp.sqrt(jnp.sum(x * x, axis=1, keepdims=True))
    xn = (x / (nrm + 1e-8)).astype(jnp.bfloat16)
    qdot = jax.lax.dot_general(
        xn, ad_ref[...].astype(jnp.bfloat16), (((1,), (0,)), ((), ())),
        preferred_element_type=jnp.float32)
    tailsum = 0.5 * jnp.sum(at_ref[...].astype(jnp.bfloat16).astype(jnp.float32))
    v = (qdot + tailsum + hb_ref[0, 0]) / R_HASH
    hq_ref[...] = jnp.mod(jnp.floor(v), TABLES)


def _query_hash(x, ad, at, hb, blk):
    rows, d = x.shape
    grid = rows // blk
    hq = pl.pallas_call(
        _qhash_kernel,
        grid=(grid,),
        in_specs=[
            pl.BlockSpec((blk, d), lambda i: (i, 0)),
            pl.BlockSpec((d, 1), lambda i: (0, 0)),
            pl.BlockSpec((8, 1), lambda i: (0, 0)),
            pl.BlockSpec((1, 1), lambda i: (0, 0)),
        ],
        out_specs=pl.BlockSpec((blk, 1), lambda i: (i, 0)),
        out_shape=jax.ShapeDtypeStruct((rows, 1), jnp.float32),
    )(x, ad, at, hb)
    return hq


def _block_qhash(x, ad_ref, at_ref, hb_ref):
    nrm = jnp.sqrt(jnp.sum(x * x, axis=1, keepdims=True))
    xn = (x / (nrm + 1e-8)).astype(jnp.bfloat16)
    qdot = jax.lax.dot_general(
        xn, ad_ref[...].astype(jnp.bfloat16), (((1,), (0,)), ((), ())),
        preferred_element_type=jnp.float32)
    tailsum = 0.5 * jnp.sum(at_ref[...].astype(jnp.bfloat16).astype(jnp.float32))
    v = (qdot + tailsum + hb_ref[0, 0]) / R_HASH
    return jnp.mod(jnp.floor(v), TABLES)


def _masked_linear_kernel(x_ref, w_ref, b_ref, ad_ref, at_ref, hb_ref,
                          hw_ref, out_ref):
    x = x_ref[...]
    hq = _block_qhash(x, ad_ref, at_ref, hb_ref)
    dense = jax.lax.dot_general(
        x, w_ref[...], (((1,), (1,)), ((), ())),
        preferred_element_type=jnp.float32)
    dense = dense + b_ref[...]
    mask = hq == hw_ref[...]
    out_ref[...] = jnp.where(mask, jnp.maximum(dense, 0.0), 0.0)


def _masked_linear(x, Wp, b, ad, at, hb, hw, bm, bn):
    B, d = x.shape
    H = Wp.shape[0]
    out = pl.pallas_call(
        _masked_linear_kernel,
        grid=(B // bm, H // bn),
        in_specs=[
            pl.BlockSpec((bm, d), lambda i, j: (i, 0)),
            pl.BlockSpec((bn, d), lambda i, j: (j, 0)),
            pl.BlockSpec((1, bn), lambda i, j: (0, j)),
            pl.BlockSpec((d, 1), lambda i, j: (0, 0)),
            pl.BlockSpec((8, 1), lambda i, j: (0, 0)),
            pl.BlockSpec((1, 1), lambda i, j: (0, 0)),
            pl.BlockSpec((1, bn), lambda i, j: (0, j)),
        ],
        out_specs=pl.BlockSpec((bm, bn), lambda i, j: (i, j)),
        out_shape=jax.ShapeDtypeStruct((B, H), jnp.float32),
    )(x, Wp, b, ad, at, hb, hw)
    return out


def _masked_linear_out_kernel(h_ref, w_ref, b_ref, ad_ref, at_ref, hb_ref,
                              hw_ref, wo_ref, bo_ref, out_ref):
    h = h_ref[...]
    hq = _block_qhash(h, ad_ref, at_ref, hb_ref)
    dense = jax.lax.dot_general(
        h.astype(jnp.bfloat16), w_ref[...].astype(jnp.bfloat16),
        (((1,), (1,)), ((), ())),
        preferred_element_type=jnp.float32)
    dense = dense + b_ref[...]
    mask = hq == hw_ref[...]
    h2 = jnp.where(mask, jnp.maximum(dense, 0.0), 0.0)
    out = jax.lax.dot_general(
        h2.astype(jnp.bfloat16), wo_ref[...].astype(jnp.bfloat16),
        (((1,), (1,)), ((), ())),
        preferred_element_type=jnp.float32)
    out_ref[...] = out + bo_ref[...]


def _masked_linear_out(h, Wp, b, ad, at, hb, hw, Wo, bo, bm):
    B, Hp = h.shape
    Op = Wo.shape[0]
    out = pl.pallas_call(
        _masked_linear_out_kernel,
        grid=(B // bm,),
        in_specs=[
            pl.BlockSpec((bm, Hp), lambda i: (i, 0)),
            pl.BlockSpec((Hp, Hp), lambda i: (0, 0)),
            pl.BlockSpec((1, Hp), lambda i: (0, 0)),
            pl.BlockSpec((Hp, 1), lambda i: (0, 0)),
            pl.BlockSpec((8, 1), lambda i: (0, 0)),
            pl.BlockSpec((1, 1), lambda i: (0, 0)),
            pl.BlockSpec((1, Hp), lambda i: (0, 0)),
            pl.BlockSpec((Op, Hp), lambda i: (0, 0)),
            pl.BlockSpec((1, Op), lambda i: (0, 0)),
        ],
        out_specs=pl.BlockSpec((bm, Op), lambda i: (i, 0)),
        out_shape=jax.ShapeDtypeStruct((B, Op), jnp.float32),
    )(h, Wp, b, ad, at, hb, hw, Wo, bo)
    return out


def kernel(x, W1, b1, W2, b2, Wout, bout, a1, hb1, a2, hb2):
    B, D = x.shape
    H = W1.shape[0]
    O = Wout.shape[0]
    Hp = 1024
    Op = 16

    W1p = jnp.pad(W1, ((0, Hp - H), (0, 0)))
    b1p = jnp.pad(b1, (0, Hp - H)).reshape(1, Hp)
    W2p = jnp.pad(W2, ((0, Hp - H), (0, Hp - H)))
    b2p = jnp.pad(b2, (0, Hp - H)).reshape(1, Hp)
    Woutp = jnp.pad(Wout, ((0, Op - O), (0, Hp - H)))
    boutp = jnp.pad(bout, (0, Op - O)).reshape(1, Op)

    a1d = a1[:D].reshape(D, 1)
    a1t = jnp.pad(a1[D:], (0, 8 - M_POW)).reshape(8, 1)
    a2d = jnp.pad(a2[:H], (0, Hp - H)).reshape(Hp, 1)
    a2t = jnp.pad(a2[H:], (0, 8 - M_POW)).reshape(8, 1)
    hb1r = hb1.reshape(1, 1)
    hb2r = hb2.reshape(1, 1)

    n1, mx1 = _row_norms(W1p, 128)
    hw1 = _weight_hash(W1p, n1, mx1, a1d, a1t, hb1r, H, 128)
    h1 = _masked_linear(x, W1p, b1p, a1d, a1t, hb1r,
                        hw1.reshape(1, Hp), 512, 1024)

    n2, mx2 = _row_norms(W2p, 128)
    hw2 = _weight_hash(W2p, n2, mx2, a2d, a2t, hb2r, H, 128)
    out = _masked_linear_out(h1, W2p, b2p, a2d, a2t, hb2r,
                             hw2.reshape(1, Hp), Woutp, boutp, 256)
    return out[:, :O]
